# baseline (speedup 1.0000x reference)
import functools

import jax
import jax.numpy as jnp
from jax import lax
from jax.experimental import pallas as pl
from jax.experimental.pallas import tpu as pltpu
from jax.experimental.pallas import tpu_sc as plsc

D = 128
B = 128
NC = 2
NS = 16
NW = NC * NS
SPLIT = 4

_mesh = functools.partial(
    plsc.VectorSubcoreMesh, core_axis_name="c", subcore_axis_name="s"
)


def _deg_call(n, bpt, npt, last):
    @functools.partial(
        pl.kernel,
        mesh=_mesh(),
        out_type=jax.ShapeDtypeStruct((NC, n, D), jnp.float32),
        scratch_types=[
            pltpu.VMEM((bpt, B), jnp.int32),
            pltpu.VMEM((B, D), jnp.float32),
            pltpu.VMEM_SHARED((NS * npt, D), jnp.float32),
        ],
    )
    def deg_kernel(dst_hbm, ones_hbm, zeros_hbm, out_hbm, dst_v, ones_v, acc):
        cid = lax.axis_index("c")
        sid = lax.axis_index("s")
        wid = cid * NS + sid
        pltpu.sync_copy(dst_hbm.at[pl.ds(wid * bpt, bpt)], dst_v)
        pltpu.sync_copy(ones_hbm, ones_v)

        @pl.when(sid < NS - 1)
        def _():
            pltpu.sync_copy(zeros_hbm, acc.at[pl.ds(sid * npt, npt)])

        @pl.when(sid == NS - 1)
        def _():
            pltpu.sync_copy(zeros_hbm.at[pl.ds(0, last)],
                            acc.at[pl.ds(sid * npt, last)])

        plsc.subcore_barrier()

        @pl.loop(0, bpt)
        def _(b):
            pltpu.sync_copy(ones_v, acc.at[dst_v.at[b]], add=True)

        plsc.subcore_barrier()

        @pl.when(sid < NS - 1)
        def _():
            pltpu.sync_copy(acc.at[pl.ds(sid * npt, npt)],
                            out_hbm.at[cid, pl.ds(sid * npt, npt)])

        @pl.when(sid == NS - 1)
        def _():
            pltpu.sync_copy(acc.at[pl.ds(sid * npt, last)],
                            out_hbm.at[cid, pl.ds(sid * npt, last)])

    return deg_kernel


def _agg_call(n, bpt, npt, last):
    PAR = B // SPLIT

    @functools.partial(
        pl.kernel,
        mesh=_mesh(),
        out_type=jax.ShapeDtypeStruct((NC, n, D), jnp.float32),
        scratch_types=[
            pltpu.VMEM((bpt * B,), jnp.int32),
            pltpu.VMEM((8, B), jnp.int32),
            pltpu.VMEM((B, D), jnp.float32),
            pltpu.VMEM((B, D), jnp.float32),
            pltpu.SemaphoreType.DMA,
            pltpu.SemaphoreType.DMA,
            pltpu.VMEM_SHARED((NS * npt, D), jnp.float32),
        ],
    )
    def agg_kernel(g_hbm, src_hbm, dst_hbm, zeros_hbm, out_hbm,
                   src_v, dst_c, r0, r1, s0, s1, acc):
        cid = lax.axis_index("c")
        sid = lax.axis_index("s")
        wid = cid * NS + sid
        pltpu.sync_copy(src_hbm.at[pl.ds(wid * bpt * B, bpt * B)], src_v)

        def fire(t, r, s):
            for p in range(SPLIT):
                pltpu.async_copy(
                    g_hbm.at[src_v.at[pl.ds(t * B + p * PAR, PAR)]],
                    r.at[pl.ds(p * PAR, PAR)], s)

        @pl.when(sid < NS - 1)
        def _():
            pltpu.sync_copy(zeros_hbm, acc.at[pl.ds(sid * npt, npt)])

        @pl.when(sid == NS - 1)
        def _():
            pltpu.sync_copy(zeros_hbm.at[pl.ds(0, last)],
                            acc.at[pl.ds(sid * npt, last)])

        plsc.subcore_barrier()

        fire(0, r0, s0)

        @pl.loop(0, bpt, step=8)
        def _(c):
            pltpu.sync_copy(dst_hbm.at[pl.ds(wid * bpt + c, 8)], dst_c)
            for jj in range(8):
                t = c + jj
                r, s = (r0, s0) if jj % 2 == 0 else (r1, s1)
                rn, sn = (r1, s1) if jj % 2 == 0 else (r0, s0)
                pltpu.make_async_copy(zeros_hbm.at[pl.ds(0, B)], r, s).wait()
                if jj == 7:
                    @pl.when(t + 1 < bpt)
                    def _():
                        fire(t + 1, rn, sn)
                else:
                    fire(t + 1, rn, sn)
                pltpu.sync_copy(r, acc.at[dst_c.at[jj]], add=True)

        plsc.subcore_barrier()

        @pl.when(sid < NS - 1)
        def _():
            pltpu.sync_copy(acc.at[pl.ds(sid * npt, npt)],
                            out_hbm.at[cid, pl.ds(sid * npt, npt)])

        @pl.when(sid == NS - 1)
        def _():
            pltpu.sync_copy(acc.at[pl.ds(sid * npt, last)],
                            out_hbm.at[cid, pl.ds(sid * npt, last)])

    return agg_kernel


def _expand_call(n, bpt, npt, last, e_pad):
    @functools.partial(
        pl.kernel,
        mesh=_mesh(),
        out_type=jax.ShapeDtypeStruct((e_pad, D), jnp.float32),
        scratch_types=[
            pltpu.VMEM((bpt * B,), jnp.int32),
            pltpu.VMEM((B, D), jnp.float32),
            pltpu.VMEM((B, D), jnp.float32),
            pltpu.SemaphoreType.DMA,
            pltpu.SemaphoreType.DMA,
            pltpu.SemaphoreType.DMA,
            pltpu.SemaphoreType.DMA,
            pltpu.VMEM_SHARED((n, D), jnp.float32),
        ],
    )
    def expand_kernel(g_hbm, src_hbm, out_hbm, src_v, r0, r1,
                      s0, s1, w0, w1, tab):
        rr = [(r0, s0, w0), (r1, s1, w1)]
        cid = lax.axis_index("c")
        sid = lax.axis_index("s")
        wid = cid * NS + sid
        pltpu.sync_copy(src_hbm.at[pl.ds(wid * bpt * B, bpt * B)], src_v)

        @pl.when(sid < NS - 1)
        def _():
            pltpu.sync_copy(g_hbm.at[pl.ds(sid * npt, npt)],
                            tab.at[pl.ds(sid * npt, npt)])

        @pl.when(sid == NS - 1)
        def _():
            pltpu.sync_copy(g_hbm.at[pl.ds(sid * npt, last)],
                            tab.at[pl.ds(sid * npt, last)])

        plsc.subcore_barrier()

        def gfire(t, r, s):
            pltpu.async_copy(tab.at[src_v.at[pl.ds(t * B, B)]], r, s)

        gfire(0, r0, s0)

        @pl.loop(0, bpt, step=2)
        def _(c):
            for jj in range(2):
                t = c + jj
                r, s, w = rr[jj]
                rn, sn, wn = rr[1 - jj]
                pltpu.make_async_copy(g_hbm.at[pl.ds(0, B)], r, s).wait()
                if jj == 0:
                    @pl.when(t > 0)
                    def _():
                        pltpu.make_async_copy(g_hbm.at[pl.ds(0, B)],
                                              rn, wn).wait()
                else:
                    pltpu.make_async_copy(g_hbm.at[pl.ds(0, B)],
                                          rn, wn).wait()
                if jj == 1:
                    @pl.when(t + 1 < bpt)
                    def _():
                        gfire(t + 1, rn, sn)
                else:
                    gfire(t + 1, rn, sn)
                pltpu.async_copy(
                    r, out_hbm.at[pl.ds((wid * bpt + t) * B, B)], w)

        pltpu.make_async_copy(g_hbm.at[pl.ds(0, B)], r1, w1).wait()

    return expand_kernel


def _reduce_call(n, bpt, npt, last, e_pad):
    @functools.partial(
        pl.kernel,
        mesh=_mesh(),
        out_type=jax.ShapeDtypeStruct((NC, n, D), jnp.float32),
        scratch_types=[
            pltpu.VMEM((bpt, B), jnp.int32),
            pltpu.VMEM((B, D), jnp.float32),
            pltpu.VMEM((B, D), jnp.float32),
            pltpu.SemaphoreType.DMA,
            pltpu.SemaphoreType.DMA,
            pltpu.VMEM_SHARED((NS * npt, D), jnp.float32),
        ],
    )
    def reduce_kernel(msgs_hbm, dst_hbm, zeros_hbm, out_hbm,
                      dst_v, r0, r1, s0, s1, acc):
        cid = lax.axis_index("c")
        sid = lax.axis_index("s")
        wid = cid * NS + sid
        pltpu.sync_copy(dst_hbm.at[pl.ds(wid * bpt, bpt)], dst_v)

        @pl.when(sid < NS - 1)
        def _():
            pltpu.sync_copy(zeros_hbm, acc.at[pl.ds(sid * npt, npt)])

        @pl.when(sid == NS - 1)
        def _():
            pltpu.sync_copy(zeros_hbm.at[pl.ds(0, last)],
                            acc.at[pl.ds(sid * npt, last)])

        plsc.subcore_barrier()

        def rfire(t, r, s):
            pltpu.async_copy(
                msgs_hbm.at[pl.ds((wid * bpt + t) * B, B)], r, s)

        rfire(0, r0, s0)

        @pl.loop(0, bpt, step=2)
        def _(c):
            for jj in range(2):
                t = c + jj
                r, s = (r0, s0) if jj == 0 else (r1, s1)
                rn, sn = (r1, s1) if jj == 0 else (r0, s0)
                pltpu.make_async_copy(msgs_hbm.at[pl.ds(0, B)], r, s).wait()
                if jj == 1:
                    @pl.when(t + 1 < bpt)
                    def _():
                        rfire(t + 1, rn, sn)
                else:
                    rfire(t + 1, rn, sn)
                pltpu.sync_copy(r, acc.at[dst_v.at[t]], add=True)

        plsc.subcore_barrier()

        @pl.when(sid < NS - 1)
        def _():
            pltpu.sync_copy(acc.at[pl.ds(sid * npt, npt)],
                            out_hbm.at[cid, pl.ds(sid * npt, npt)])

        @pl.when(sid == NS - 1)
        def _():
            pltpu.sync_copy(acc.at[pl.ds(sid * npt, last)],
                            out_hbm.at[cid, pl.ds(sid * npt, last)])

    return reduce_kernel


def _dinv_from(degp):
    return lax.rsqrt(degp[0, :, 0:1] + degp[1, :, 0:1] + 1.0)


def _mm_body(x_ref, w_ref, o_ref):
    o_ref[...] = jnp.dot(x_ref[...], w_ref[...],
                         preferred_element_type=jnp.float32)


def _scale_body(h_ref, degp_ref, o_ref):
    o_ref[...] = h_ref[...] * _dinv_from(degp_ref[...])


def _layer2_body(aggp_ref, g1_ref, degp_ref, b1_ref, w2_ref, o_ref):
    dinv = _dinv_from(degp_ref[...])
    a = aggp_ref[...]
    t = dinv * (a[0] + a[1] + g1_ref[...]) + b1_ref[...]
    t = jnp.maximum(t, 0.0)
    o_ref[...] = jnp.dot(t, w2_ref[...],
                         preferred_element_type=jnp.float32) * dinv


def _final_body(aggp_ref, g2_ref, degp_ref, b2_ref, o_ref):
    dinv = _dinv_from(degp_ref[...])
    a = aggp_ref[...]
    o_ref[...] = dinv * (a[0] + a[1] + g2_ref[...]) + b2_ref[...]


def _row_spec(bm, w):
    return pl.BlockSpec((bm, w), lambda i: (i, 0))


def _part_spec(bm, w):
    return pl.BlockSpec((NC, bm, w), lambda i: (0, i, 0))


def _full_spec(r, c):
    return pl.BlockSpec((r, c), lambda i: (0, 0))


def kernel(x, edge_index, W1, b1, W2, b2):
    n = x.shape[0]
    e = edge_index.shape[1]
    bpt = -(-e // (NW * B))
    bpt = -(-bpt // 8) * 8
    e_pad = NW * bpt * B
    npt = -(-(-(-n // NS)) // 8) * 8
    last = n - (NS - 1) * npt
    assert 0 < last <= npt and NS * npt > n

    src = edge_index[0]
    dst = edge_index[1]
    pad = e_pad - e
    src_p = jnp.concatenate([src, jnp.zeros((pad,), src.dtype)])
    dst_p = jnp.concatenate([dst, jnp.full((pad,), n, dst.dtype)])
    dst_p = dst_p.reshape(NW * bpt, B)

    ones_bd = jnp.ones((B, D), jnp.float32)
    zeros_nd = jnp.zeros((npt, D), jnp.float32)
    assert last % 8 == 0

    bm = 2000
    assert n % bm == 0
    grid = (n // bm,)

    matmul = pl.pallas_call(
        _mm_body,
        grid=grid,
        in_specs=[_row_spec(bm, D), _full_spec(D, D)],
        out_specs=_row_spec(bm, D),
        out_shape=jax.ShapeDtypeStruct((n, D), jnp.float32),
    )
    scale = pl.pallas_call(
        _scale_body,
        grid=grid,
        in_specs=[_row_spec(bm, D), _part_spec(bm, D)],
        out_specs=_row_spec(bm, D),
        out_shape=jax.ShapeDtypeStruct((n, D), jnp.float32),
    )
    layer2 = pl.pallas_call(
        _layer2_body,
        grid=grid,
        in_specs=[_part_spec(bm, D), _row_spec(bm, D), _part_spec(bm, D),
                  _full_spec(1, D), _full_spec(D, D)],
        out_specs=_row_spec(bm, D),
        out_shape=jax.ShapeDtypeStruct((n, D), jnp.float32),
    )
    final = pl.pallas_call(
        _final_body,
        grid=grid,
        in_specs=[_part_spec(bm, D), _row_spec(bm, D), _part_spec(bm, D),
                  _full_spec(1, D)],
        out_specs=_row_spec(bm, D),
        out_shape=jax.ShapeDtypeStruct((n, D), jnp.float32),
    )

    deg = _deg_call(n, bpt, npt, last)
    expand = _expand_call(n, bpt, npt, last, e_pad)
    reduce_ = _reduce_call(n, bpt, npt, last, e_pad)

    def agg(g):
        return reduce_(expand(g, src_p), dst_p, zeros_nd)

    degp = deg(dst_p, ones_bd, zeros_nd)
    h1 = matmul(x, W1)
    g1 = scale(h1, degp)
    a1 = agg(g1)
    g2 = layer2(a1, g1, degp, b1.reshape(1, D), W2)
    a2 = agg(g2)
    out = final(a2, g2, degp, b2.reshape(1, D))
    return out


# speedup vs baseline: 1.1136x; 1.1136x over previous
import dataclasses
import functools

import jax
import jax.numpy as jnp
from jax import lax
from jax.experimental import pallas as pl
from jax.experimental.pallas import tpu as pltpu
from jax.experimental.pallas import tpu_sc as plsc

D = 128
B = 128
NC = 2
NS = 16
NW = NC * NS
SPLIT = 4

_mesh = functools.partial(
    plsc.VectorSubcoreMesh, core_axis_name="c", subcore_axis_name="s"
)


def _no_layout_params():
    cp = pltpu.CompilerParams()
    if "needs_layout_passes" in pltpu.CompilerParams.__dataclass_fields__:
        cp = dataclasses.replace(cp, needs_layout_passes=False)
    return cp


def _deg_call(n, bpt, npt, last):
    HR = -(-(n + 1) // (128 * 8)) * 8

    @functools.partial(
        pl.kernel,
        mesh=_mesh(),
        compiler_params=_no_layout_params(),
        out_type=jax.ShapeDtypeStruct((NC, HR, 128), jnp.float32),
        scratch_types=[
            pltpu.VMEM((bpt * B,), jnp.int32),
            pltpu.VMEM((HR, 128), jnp.float32),
            pltpu.VMEM((8, 128), jnp.float32),
            pltpu.VMEM((8, 128), jnp.float32),
            pltpu.VMEM_SHARED((NS, HR, 128), jnp.float32),
        ],
    )
    def deg_kernel(dst_hbm, out_hbm, dst_v, hist, tmp, accv, stage):
        cid = lax.axis_index("c")
        sid = lax.axis_index("s")
        wid = cid * NS + sid
        pltpu.sync_copy(dst_hbm.at[pl.ds(wid * bpt * B, bpt * B)], dst_v)

        zero16 = jnp.zeros((16,), jnp.float32)
        one16 = jnp.ones((16,), jnp.float32)

        @pl.loop(0, HR)
        def _(r):
            for c in range(8):
                hist[r, pl.ds(c * 16, 16)] = zero16

        @pl.loop(0, bpt * B // 16)
        def _(k):
            idx = dst_v[pl.ds(k * 16, 16)]
            plsc.addupdate_scatter(
                hist, [jnp.right_shift(idx, 7), jnp.bitwise_and(idx, 127)],
                one16)

        pltpu.sync_copy(hist, stage.at[sid])
        plsc.subcore_barrier()

        @pl.when(sid < HR // 8)
        def _():
            for r in range(8):
                for c in range(8):
                    accv[r, pl.ds(c * 16, 16)] = zero16

            @pl.loop(0, NS)
            def _(t):
                pltpu.sync_copy(stage.at[t, pl.ds(sid * 8, 8)], tmp)
                for r in range(8):
                    for c in range(8):
                        sl = (r, pl.ds(c * 16, 16))
                        accv[sl] = accv[sl] + tmp[sl]

            pltpu.sync_copy(accv, out_hbm.at[cid, pl.ds(sid * 8, 8)])

    return deg_kernel


def _expand_call(n, bpt, npt, last, e_pad):
    @functools.partial(
        pl.kernel,
        mesh=_mesh(),
        out_type=jax.ShapeDtypeStruct((e_pad, D), jnp.float32),
        scratch_types=[
            pltpu.VMEM((bpt * B,), jnp.int32),
            pltpu.VMEM((B, D), jnp.float32),
            pltpu.VMEM((B, D), jnp.float32),
            pltpu.SemaphoreType.DMA,
            pltpu.SemaphoreType.DMA,
            pltpu.SemaphoreType.DMA,
            pltpu.SemaphoreType.DMA,
            pltpu.VMEM_SHARED((n, D), jnp.float32),
        ],
    )
    def expand_kernel(g_hbm, src_hbm, out_hbm, src_v, r0, r1,
                      s0, s1, w0, w1, tab):
        rr = [(r0, s0, w0), (r1, s1, w1)]
        cid = lax.axis_index("c")
        sid = lax.axis_index("s")
        wid = cid * NS + sid
        pltpu.sync_copy(src_hbm.at[pl.ds(wid * bpt * B, bpt * B)], src_v)

        @pl.when(sid < NS - 1)
        def _():
            pltpu.sync_copy(g_hbm.at[pl.ds(sid * npt, npt)],
                            tab.at[pl.ds(sid * npt, npt)])

        @pl.when(sid == NS - 1)
        def _():
            pltpu.sync_copy(g_hbm.at[pl.ds(sid * npt, last)],
                            tab.at[pl.ds(sid * npt, last)])

        plsc.subcore_barrier()

        def gfire(t, r, s):
            pltpu.async_copy(tab.at[src_v.at[pl.ds(t * B, B)]], r, s)

        gfire(0, r0, s0)

        @pl.loop(0, bpt, step=2)
        def _(c):
            for jj in range(2):
                t = c + jj
                r, s, w = rr[jj]
                rn, sn, wn = rr[1 - jj]
                pltpu.make_async_copy(g_hbm.at[pl.ds(0, B)], r, s).wait()
                if jj == 0:
                    @pl.when(t > 0)
                    def _():
                        pltpu.make_async_copy(g_hbm.at[pl.ds(0, B)],
                                              rn, wn).wait()
                else:
                    pltpu.make_async_copy(g_hbm.at[pl.ds(0, B)],
                                          rn, wn).wait()
                if jj == 1:
                    @pl.when(t + 1 < bpt)
                    def _():
                        gfire(t + 1, rn, sn)
                else:
                    gfire(t + 1, rn, sn)
                pltpu.async_copy(
                    r, out_hbm.at[pl.ds((wid * bpt + t) * B, B)], w)

        pltpu.make_async_copy(g_hbm.at[pl.ds(0, B)], r1, w1).wait()

    return expand_kernel


def _reduce_call(n, bpt, npt, last, e_pad):
    @functools.partial(
        pl.kernel,
        mesh=_mesh(),
        out_type=jax.ShapeDtypeStruct((NC, n, D), jnp.float32),
        scratch_types=[
            pltpu.VMEM((bpt, B), jnp.int32),
            pltpu.VMEM((B, D), jnp.float32),
            pltpu.VMEM((B, D), jnp.float32),
            pltpu.SemaphoreType.DMA,
            pltpu.SemaphoreType.DMA,
            pltpu.VMEM_SHARED((NS * npt, D), jnp.float32),
        ],
    )
    def reduce_kernel(msgs_hbm, dst_hbm, zeros_hbm, out_hbm,
                      dst_v, r0, r1, s0, s1, acc):
        cid = lax.axis_index("c")
        sid = lax.axis_index("s")
        wid = cid * NS + sid
        pltpu.sync_copy(dst_hbm.at[pl.ds(wid * bpt, bpt)], dst_v)

        @pl.when(sid < NS - 1)
        def _():
            pltpu.sync_copy(zeros_hbm, acc.at[pl.ds(sid * npt, npt)])

        @pl.when(sid == NS - 1)
        def _():
            pltpu.sync_copy(zeros_hbm.at[pl.ds(0, last)],
                            acc.at[pl.ds(sid * npt, last)])

        plsc.subcore_barrier()

        def rfire(t, r, s):
            pltpu.async_copy(
                msgs_hbm.at[pl.ds((wid * bpt + t) * B, B)], r, s)

        rfire(0, r0, s0)

        @pl.loop(0, bpt, step=2)
        def _(c):
            for jj in range(2):
                t = c + jj
                r, s = (r0, s0) if jj == 0 else (r1, s1)
                rn, sn = (r1, s1) if jj == 0 else (r0, s0)
                pltpu.make_async_copy(msgs_hbm.at[pl.ds(0, B)], r, s).wait()
                if jj == 1:
                    @pl.when(t + 1 < bpt)
                    def _():
                        rfire(t + 1, rn, sn)
                else:
                    rfire(t + 1, rn, sn)
                pltpu.sync_copy(r, acc.at[dst_v.at[t]], add=True)

        plsc.subcore_barrier()

        @pl.when(sid < NS - 1)
        def _():
            pltpu.sync_copy(acc.at[pl.ds(sid * npt, npt)],
                            out_hbm.at[cid, pl.ds(sid * npt, npt)])

        @pl.when(sid == NS - 1)
        def _():
            pltpu.sync_copy(acc.at[pl.ds(sid * npt, last)],
                            out_hbm.at[cid, pl.ds(sid * npt, last)])

    return reduce_kernel


def _dinv_from(dcol):
    return lax.rsqrt(dcol + 1.0)


def _mm_scale_body(x_ref, w_ref, dcol_ref, o_ref):
    o_ref[...] = jnp.dot(x_ref[...], w_ref[...],
                         preferred_element_type=jnp.float32) * _dinv_from(
                             dcol_ref[...])


def _layer2_body(aggp_ref, g1_ref, dcol_ref, b1_ref, w2_ref, o_ref):
    dinv = _dinv_from(dcol_ref[...])
    a = aggp_ref[...]
    t = dinv * (a[0] + a[1] + g1_ref[...]) + b1_ref[...]
    t = jnp.maximum(t, 0.0)
    o_ref[...] = jnp.dot(t, w2_ref[...],
                         preferred_element_type=jnp.float32) * dinv


def _final_body(aggp_ref, g2_ref, dcol_ref, b2_ref, o_ref):
    dinv = _dinv_from(dcol_ref[...])
    a = aggp_ref[...]
    o_ref[...] = dinv * (a[0] + a[1] + g2_ref[...]) + b2_ref[...]


def _row_spec(bm, w):
    return pl.BlockSpec((bm, w), lambda i: (i, 0))


def _part_spec(bm, w):
    return pl.BlockSpec((NC, bm, w), lambda i: (0, i, 0))


def _full_spec(r, c):
    return pl.BlockSpec((r, c), lambda i: (0, 0))


def kernel(x, edge_index, W1, b1, W2, b2):
    n = x.shape[0]
    e = edge_index.shape[1]
    bpt = -(-e // (NW * B))
    bpt = -(-bpt // 8) * 8
    e_pad = NW * bpt * B
    npt = -(-(-(-n // NS)) // 8) * 8
    last = n - (NS - 1) * npt
    assert 0 < last <= npt and NS * npt > n

    src = edge_index[0]
    dst = edge_index[1]
    pad = e_pad - e
    src_p = jnp.concatenate([src, jnp.zeros((pad,), src.dtype)])
    dst_p = jnp.concatenate([dst, jnp.full((pad,), n, dst.dtype)])
    dst_p = dst_p.reshape(NW * bpt, B)

    zeros_nd = jnp.zeros((npt, D), jnp.float32)
    assert last % 8 == 0

    bm = 2000
    assert n % bm == 0
    grid = (n // bm,)

    mm_scale = pl.pallas_call(
        _mm_scale_body,
        grid=grid,
        in_specs=[_row_spec(bm, D), _full_spec(D, D), _row_spec(bm, 1)],
        out_specs=_row_spec(bm, D),
        out_shape=jax.ShapeDtypeStruct((n, D), jnp.float32),
    )
    layer2 = pl.pallas_call(
        _layer2_body,
        grid=grid,
        in_specs=[_part_spec(bm, D), _row_spec(bm, D), _row_spec(bm, 1),
                  _full_spec(1, D), _full_spec(D, D)],
        out_specs=_row_spec(bm, D),
        out_shape=jax.ShapeDtypeStruct((n, D), jnp.float32),
    )
    final = pl.pallas_call(
        _final_body,
        grid=grid,
        in_specs=[_part_spec(bm, D), _row_spec(bm, D), _row_spec(bm, 1),
                  _full_spec(1, D)],
        out_specs=_row_spec(bm, D),
        out_shape=jax.ShapeDtypeStruct((n, D), jnp.float32),
    )

    deg = _deg_call(n, bpt, npt, last)
    expand = _expand_call(n, bpt, npt, last, e_pad)
    reduce_ = _reduce_call(n, bpt, npt, last, e_pad)

    def agg(g):
        return reduce_(expand(g, src_p), dst_p, zeros_nd)

    dst_flat = dst_p.reshape(-1)
    degp = deg(dst_flat)
    dcol = (degp[0] + degp[1]).reshape(-1)[:n, None]
    g1 = mm_scale(x, W1, dcol)
    a1 = agg(g1)
    g2 = layer2(a1, g1, dcol, b1.reshape(1, D), W2)
    a2 = agg(g2)
    out = final(a2, g2, dcol, b2.reshape(1, D))
    return out
